# baseline (speedup 1.0000x reference)
import functools

import jax
import jax.numpy as jnp
from jax import lax
from jax.experimental import pallas as pl
from jax.experimental.pallas import tpu as pltpu
from jax.experimental.pallas import tpu_sc as plsc

E = 16
D = 2048
H = 2048
H2 = 1024
C = 100
CP = 128
B = 256
NK = 4
CK = D // NK
NEG_INF = float('-inf')


def _gate_body(feat_ref, w1_ref, b1_ref, w2_ref, b2_ref, logits_ref, acc_ref):
    i = pl.program_id(0)
    part = jnp.dot(feat_ref[...].astype(jnp.bfloat16),
                   w1_ref[...].astype(jnp.bfloat16),
                   preferred_element_type=jnp.float32)

    @pl.when(i == 0)
    def _():
        acc_ref[...] = part

    @pl.when(i > 0)
    def _():
        acc_ref[...] += part

    @pl.when(i == NK - 1)
    def _():
        g_hid = jnp.maximum(acc_ref[...] + b1_ref[...], 0.0)
        logits_ref[...] = jnp.dot(
            g_hid.astype(jnp.bfloat16), w2_ref[...].astype(jnp.bfloat16),
            preferred_element_type=jnp.float32) + b2_ref[...]


def _gate_logits(feat, wg1, bg1, wg2, bg2):
    return pl.pallas_call(
        _gate_body,
        grid=(NK,),
        in_specs=[
            pl.BlockSpec((B, CK), lambda i: (0, i)),
            pl.BlockSpec((CK, D), lambda i: (i, 0)),
            pl.BlockSpec((1, D), lambda i: (0, 0)),
            pl.BlockSpec((D, E), lambda i: (0, 0)),
            pl.BlockSpec((1, E), lambda i: (0, 0)),
        ],
        out_specs=pl.BlockSpec((B, E), lambda i: (0, 0)),
        out_shape=jax.ShapeDtypeStruct((B, E), jnp.float32),
        scratch_shapes=[pltpu.VMEM((B, D), jnp.float32)],
    )(feat, wg1, bg1.reshape(1, D), wg2, bg2.reshape(1, E))


def _expert_body(feat_ref, w1_ref, b1_ref, w2_ref, b2_ref, w3_ref, b3_ref,
                 o_ref):
    feat = feat_ref[...].astype(jnp.bfloat16)
    h1 = jnp.dot(feat, w1_ref[0].astype(jnp.bfloat16),
                 preferred_element_type=jnp.float32)
    h1 = jnp.maximum(h1 + b1_ref[0], 0.0).astype(jnp.bfloat16)
    h2 = jnp.dot(h1, w2_ref[0].astype(jnp.bfloat16),
                 preferred_element_type=jnp.float32)
    h2 = jnp.maximum(h2 + b2_ref[0], 0.0).astype(jnp.bfloat16)
    out = jnp.dot(h2, w3_ref[0].astype(jnp.bfloat16),
                  preferred_element_type=jnp.float32)
    o_ref[0] = out + b3_ref[0]


def _experts(feat, w1, b1, w2, b2, w3p, b3p):
    return pl.pallas_call(
        _expert_body,
        grid=(E,),
        in_specs=[
            pl.BlockSpec((B, D), lambda e: (0, 0)),
            pl.BlockSpec((1, D, H), lambda e: (e, 0, 0)),
            pl.BlockSpec((1, 1, H), lambda e: (e, 0, 0)),
            pl.BlockSpec((1, H, H2), lambda e: (e, 0, 0)),
            pl.BlockSpec((1, 1, H2), lambda e: (e, 0, 0)),
            pl.BlockSpec((1, H2, CP), lambda e: (e, 0, 0)),
            pl.BlockSpec((1, 1, CP), lambda e: (e, 0, 0)),
        ],
        out_specs=pl.BlockSpec((1, B, CP), lambda e: (e, 0, 0)),
        out_shape=jax.ShapeDtypeStruct((E, B, CP), jnp.float32),
    )(feat, w1, b1, w2, b2, w3p, b3p)


def _bfly_max(v, lane):
    for k in (1, 2, 4, 8):
        v = jnp.maximum(v, v.at[lane ^ k].get(mode='promise_in_bounds'))
    return v


def _bfly_min_i32(v, lane):
    for k in (1, 2, 4, 8):
        v = jnp.minimum(v, v.at[lane ^ k].get(mode='promise_in_bounds'))
    return v


def _bfly_sum(v, lane):
    for k in (1, 2, 4, 8):
        v = v + v.at[lane ^ k].get(mode='promise_in_bounds')
    return v


def _lowest_lane(mask, lane):
    return _bfly_min_i32(jnp.where(mask, lane, jnp.int32(64)), lane)


def _top2_row(row, lane):
    m1 = _bfly_max(row, lane)
    i1 = _lowest_lane(row == m1, lane)
    row2 = jnp.where(lane == i1, NEG_INF, row)
    m2 = _bfly_max(row2, lane)
    i2 = _lowest_lane(row2 == m2, lane)
    e2 = jnp.exp(m2 - m1)
    den = 1.0 + e2
    g1 = 1.0 / den
    g2 = e2 / den
    zero = jnp.zeros((16,), jnp.float32)
    gates_row = (jnp.where(lane == i1, g1, zero)
                 + jnp.where(lane == i2, g2, zero))
    return i1, i2, g1, g2, gates_row


def _sc_body(logits_hbm, oute_hbm, y_hbm, loss_hbm,
             lg_v, lg_all, idx_v, gbuf, rows_v, y_v, loss_v, sem):
    cid = lax.axis_index("c")
    sid = lax.axis_index("s")
    wid = sid * 2 + cid
    base = wid * 8
    lane = lax.iota(jnp.int32, 16)

    pltpu.sync_copy(logits_hbm.at[pl.ds(base, 8)], lg_v)

    idx_vec = jnp.zeros((16,), jnp.int32)
    for t in range(8):
        row = lg_v[t, pl.ds(0, 16)]
        i1, i2, g1, g2, _ = _top2_row(row, lane)
        b = base + t
        idx_vec = jnp.where(lane == 2 * t, i1 * B + b, idx_vec)
        idx_vec = jnp.where(lane == 2 * t + 1, i2 * B + b, idx_vec)
        gbuf[2 * t] = g1
        gbuf[2 * t + 1] = g2
    idx_v[...] = idx_vec

    pltpu.async_copy(oute_hbm.at[idx_v], rows_v, sem).wait()

    for t in range(8):
        g1 = gbuf[2 * t]
        g2 = gbuf[2 * t + 1]
        for c in range(CP // 16):
            r1 = rows_v[2 * t, pl.ds(c * 16, 16)]
            r2 = rows_v[2 * t + 1, pl.ds(c * 16, 16)]
            y_v[t, pl.ds(c * 16, 16)] = g1 * r1 + g2 * r2
    pltpu.sync_copy(y_v, y_hbm.at[pl.ds(base, 8)])

    @pl.when(jnp.logical_and(cid == 0, sid == 0))
    def _loss():
        pltpu.sync_copy(logits_hbm, lg_all)

        def step(i, carry):
            imp, ld = carry
            row = lg_all[i, pl.ds(0, 16)]
            _, _, _, _, gates_row = _top2_row(row, lane)
            imp = imp + gates_row
            ld = ld + jnp.where(gates_row > 0.0,
                                jnp.full((16,), 1.0, jnp.float32),
                                jnp.zeros((16,), jnp.float32))
            return imp, ld

        zero = jnp.zeros((16,), jnp.float32)
        imp, ld = lax.fori_loop(0, B, step, (zero, zero))

        def cv_sq(v):
            mean = _bfly_sum(v, lane) * (1.0 / 16.0)
            d = v - mean
            var = _bfly_sum(d * d, lane) * (1.0 / 15.0)
            return var / (mean * mean + 1e-10)

        loss = (cv_sq(imp) + cv_sq(ld)) * 1e-2
        loss_v[...] = loss
        pltpu.sync_copy(loss_v, loss_hbm)


def _routing_combine(logits, oute_flat):
    mesh = plsc.VectorSubcoreMesh(core_axis_name="c", subcore_axis_name="s")
    fn = functools.partial(
        pl.kernel,
        mesh=mesh,
        out_type=[
            jax.ShapeDtypeStruct((B, CP), jnp.float32),
            jax.ShapeDtypeStruct((16,), jnp.float32),
        ],
        scratch_types=[
            pltpu.VMEM((8, E), jnp.float32),
            pltpu.VMEM((B, E), jnp.float32),
            pltpu.VMEM((16,), jnp.int32),
            pltpu.VMEM((16, 16), jnp.float32),
            pltpu.VMEM((16, CP), jnp.float32),
            pltpu.VMEM((8, CP), jnp.float32),
            pltpu.VMEM((16,), jnp.float32),
            pltpu.SemaphoreType.DMA,
        ],
    )(_sc_body)
    return fn(logits, oute_flat)


def _conv_body(Hh, Ww, x_ref, w_ref, s_ref, t_ref, o_ref):
    k = x_ref.shape[0]
    Ci = x_ref.shape[3]
    Co = w_ref.shape[2]
    acc = jnp.zeros((k * Hh * Ww, Co), jnp.float32)
    for dh in range(3):
        for dw in range(3):
            xs = x_ref[:, dh:dh + Hh, dw:dw + Ww, :]
            xs2 = xs.reshape(k * Hh * Ww, Ci).astype(jnp.bfloat16)
            acc = acc + jnp.dot(xs2, w_ref[dh * 3 + dw].astype(jnp.bfloat16),
                                preferred_element_type=jnp.float32)
    y = jnp.maximum(acc * s_ref[0] + t_ref[0], 0.0)
    y6 = y.reshape(k, Hh // 2, 2, Ww // 2, 2, Co)
    pooled = jnp.max(jnp.max(y6, axis=4), axis=2)
    if o_ref.shape[1] == Hh // 2:
        o_ref[...] = pooled
    else:
        o_ref[...] = jnp.zeros_like(o_ref)
        o_ref[:, 1:1 + Hh // 2, 1:1 + Ww // 2, :] = pooled


def _conv_layer(xp, w9, s, t, k, last=False):
    Bn, Hp, Wp, Ci = xp.shape
    Hh, Ww = Hp - 2, Wp - 2
    Co = w9.shape[2]
    if last:
        Ho, Wo = Hh // 2, Ww // 2
    else:
        Ho, Wo = Hh // 2 + 2, Ww // 2 + 2
    body = functools.partial(_conv_body, Hh, Ww)
    return pl.pallas_call(
        body,
        grid=(Bn // k,),
        in_specs=[
            pl.BlockSpec((k, Hp, Wp, Ci), lambda i: (i, 0, 0, 0)),
            pl.BlockSpec((9, Ci, Co), lambda i: (0, 0, 0)),
            pl.BlockSpec((1, Co), lambda i: (0, 0)),
            pl.BlockSpec((1, Co), lambda i: (0, 0)),
        ],
        out_specs=pl.BlockSpec((k, Ho, Wo, Co), lambda i: (i, 0, 0, 0)),
        out_shape=jax.ShapeDtypeStruct((Bn, Ho, Wo, Co), jnp.float32),
    )(xp, w9, s, t)


def _trunk(x, p):
    eps = 1e-5
    h = jnp.transpose(x, (0, 2, 3, 1))
    h = jnp.pad(h, ((0, 0), (1, 1), (1, 1), (0, 0)))
    ks = (8, 8, 16, 32)
    for i in range(1, 5):
        w = p['conv%d_w' % i]
        w9 = jnp.transpose(w, (2, 3, 1, 0)).reshape(9, w.shape[1], w.shape[0])
        g = p['bn%d_g' % i] / jnp.sqrt(1.0 + eps)
        s = g[None, :]
        t = (g * p['conv%d_b' % i] + p['bn%d_b' % i])[None, :]
        h = _conv_layer(h, w9, s, t, ks[i - 1], last=(i == 4))
    feat = jnp.transpose(h, (0, 3, 1, 2)).reshape(x.shape[0], D)
    return feat


def kernel(x, params):
    p = params
    feat = _trunk(x, p)

    logits = _gate_logits(feat, p['wg1'], p['bg1'], p['wg2'], p['bg2'])

    w3p = jnp.pad(p['ew3'], ((0, 0), (0, 0), (0, CP - C)))
    b3p = jnp.pad(p['eb3'], ((0, 0), (0, CP - C))).reshape(E, 1, CP)
    oute = _experts(feat,
                    p['ew1'], p['eb1'].reshape(E, 1, H),
                    p['ew2'], p['eb2'].reshape(E, 1, H2),
                    w3p, b3p)

    y_pad, loss_vec = _routing_combine(logits, oute.reshape(E * B, CP))
    return y_pad[:, :C], loss_vec[0].reshape(())


# speedup vs baseline: 2.6271x; 2.6271x over previous
import functools

import jax
import jax.numpy as jnp
from jax import lax
from jax.experimental import pallas as pl
from jax.experimental.pallas import tpu as pltpu
from jax.experimental.pallas import tpu_sc as plsc

E = 16
D = 2048
H = 2048
H2 = 1024
C = 100
CP = 128
B = 256
NK = 4
CK = D // NK
NEG_INF = float('-inf')


def _gate_body(feat_ref, w1_ref, b1_ref, w2_ref, b2_ref, logits_ref, acc_ref):
    i = pl.program_id(0)
    part = jnp.dot(feat_ref[...].astype(jnp.bfloat16),
                   w1_ref[...].astype(jnp.bfloat16),
                   preferred_element_type=jnp.float32)

    @pl.when(i == 0)
    def _():
        acc_ref[...] = part

    @pl.when(i > 0)
    def _():
        acc_ref[...] += part

    @pl.when(i == NK - 1)
    def _():
        g_hid = jnp.maximum(acc_ref[...] + b1_ref[...], 0.0)
        logits_ref[...] = jnp.dot(
            g_hid.astype(jnp.bfloat16), w2_ref[...].astype(jnp.bfloat16),
            preferred_element_type=jnp.float32) + b2_ref[...]


def _gate_logits(feat, wg1, bg1, wg2, bg2):
    return pl.pallas_call(
        _gate_body,
        grid=(NK,),
        in_specs=[
            pl.BlockSpec((B, CK), lambda i: (0, i)),
            pl.BlockSpec((CK, D), lambda i: (i, 0)),
            pl.BlockSpec((1, D), lambda i: (0, 0)),
            pl.BlockSpec((D, E), lambda i: (0, 0)),
            pl.BlockSpec((1, E), lambda i: (0, 0)),
        ],
        out_specs=pl.BlockSpec((B, E), lambda i: (0, 0)),
        out_shape=jax.ShapeDtypeStruct((B, E), jnp.float32),
        scratch_shapes=[pltpu.VMEM((B, D), jnp.float32)],
    )(feat, wg1, bg1.reshape(1, D), wg2, bg2.reshape(1, E))


def _expert_body(feat_ref, w1_ref, b1_ref, w2_ref, b2_ref, w3_ref, b3_ref,
                 o_ref):
    feat = feat_ref[...].astype(jnp.bfloat16)
    h1 = jnp.dot(feat, w1_ref[0].astype(jnp.bfloat16),
                 preferred_element_type=jnp.float32)
    h1 = jnp.maximum(h1 + b1_ref[0], 0.0).astype(jnp.bfloat16)
    h2 = jnp.dot(h1, w2_ref[0].astype(jnp.bfloat16),
                 preferred_element_type=jnp.float32)
    h2 = jnp.maximum(h2 + b2_ref[0], 0.0).astype(jnp.bfloat16)
    out = jnp.dot(h2, w3_ref[0].astype(jnp.bfloat16),
                  preferred_element_type=jnp.float32)
    o_ref[0] = out + b3_ref[0]


def _experts(feat, w1, b1, w2, b2, w3p, b3p):
    return pl.pallas_call(
        _expert_body,
        grid=(E,),
        in_specs=[
            pl.BlockSpec((B, D), lambda e: (0, 0)),
            pl.BlockSpec((1, D, H), lambda e: (e, 0, 0)),
            pl.BlockSpec((1, 1, H), lambda e: (e, 0, 0)),
            pl.BlockSpec((1, H, H2), lambda e: (e, 0, 0)),
            pl.BlockSpec((1, 1, H2), lambda e: (e, 0, 0)),
            pl.BlockSpec((1, H2, CP), lambda e: (e, 0, 0)),
            pl.BlockSpec((1, 1, CP), lambda e: (e, 0, 0)),
        ],
        out_specs=pl.BlockSpec((1, B, CP), lambda e: (e, 0, 0)),
        out_shape=jax.ShapeDtypeStruct((E, B, CP), jnp.float32),
    )(feat, w1, b1, w2, b2, w3p, b3p)


def _bfly_max(v, lane):
    for k in (1, 2, 4, 8):
        v = jnp.maximum(v, v.at[lane ^ k].get(mode='promise_in_bounds'))
    return v


def _bfly_min_i32(v, lane):
    for k in (1, 2, 4, 8):
        v = jnp.minimum(v, v.at[lane ^ k].get(mode='promise_in_bounds'))
    return v


def _bfly_sum(v, lane):
    for k in (1, 2, 4, 8):
        v = v + v.at[lane ^ k].get(mode='promise_in_bounds')
    return v


def _lowest_lane(mask, lane):
    return _bfly_min_i32(jnp.where(mask, lane, jnp.int32(64)), lane)


def _top2_row(row, lane):
    m1 = _bfly_max(row, lane)
    i1 = _lowest_lane(row == m1, lane)
    row2 = jnp.where(lane == i1, NEG_INF, row)
    m2 = _bfly_max(row2, lane)
    i2 = _lowest_lane(row2 == m2, lane)
    e2 = jnp.exp(m2 - m1)
    den = 1.0 + e2
    g1 = 1.0 / den
    g2 = e2 / den
    zero = jnp.zeros((16,), jnp.float32)
    gates_row = (jnp.where(lane == i1, g1, zero)
                 + jnp.where(lane == i2, g2, zero))
    return i1, i2, g1, g2, gates_row


def _sc_body(logits_hbm, oute_hbm, y_hbm, loss_hbm,
             lg_v, lg_all, idx_v, gbuf, rows_v, y_v, loss_v, sem):
    cid = lax.axis_index("c")
    sid = lax.axis_index("s")
    wid = sid * 2 + cid
    base = wid * 8
    lane = lax.iota(jnp.int32, 16)

    pltpu.sync_copy(logits_hbm.at[pl.ds(base, 8)], lg_v)

    idx_vec = jnp.zeros((16,), jnp.int32)
    for t in range(8):
        row = lg_v[t, pl.ds(0, 16)]
        i1, i2, g1, g2, _ = _top2_row(row, lane)
        b = base + t
        idx_vec = jnp.where(lane == 2 * t, i1 * B + b, idx_vec)
        idx_vec = jnp.where(lane == 2 * t + 1, i2 * B + b, idx_vec)
        gbuf[2 * t] = g1
        gbuf[2 * t + 1] = g2
    idx_v[...] = idx_vec

    pltpu.async_copy(oute_hbm.at[idx_v], rows_v, sem).wait()

    for t in range(8):
        g1 = gbuf[2 * t]
        g2 = gbuf[2 * t + 1]
        for c in range(CP // 16):
            r1 = rows_v[2 * t, pl.ds(c * 16, 16)]
            r2 = rows_v[2 * t + 1, pl.ds(c * 16, 16)]
            y_v[t, pl.ds(c * 16, 16)] = g1 * r1 + g2 * r2
    pltpu.sync_copy(y_v, y_hbm.at[pl.ds(base, 8)])

    @pl.when(jnp.logical_and(cid == 0, sid == 0))
    def _loss():
        pltpu.sync_copy(logits_hbm, lg_all)

        def step(i, carry):
            imp, ld = carry
            row = lg_all[i, pl.ds(0, 16)]
            _, _, _, _, gates_row = _top2_row(row, lane)
            imp = imp + gates_row
            ld = ld + jnp.where(gates_row > 0.0,
                                jnp.full((16,), 1.0, jnp.float32),
                                jnp.zeros((16,), jnp.float32))
            return imp, ld

        zero = jnp.zeros((16,), jnp.float32)
        imp, ld = lax.fori_loop(0, B, step, (zero, zero))

        def cv_sq(v):
            mean = _bfly_sum(v, lane) * (1.0 / 16.0)
            d = v - mean
            var = _bfly_sum(d * d, lane) * (1.0 / 15.0)
            return var / (mean * mean + 1e-10)

        loss = (cv_sq(imp) + cv_sq(ld)) * 1e-2
        loss_v[...] = loss
        pltpu.sync_copy(loss_v, loss_hbm)


def _routing_combine(logits, oute_flat):
    mesh = plsc.VectorSubcoreMesh(core_axis_name="c", subcore_axis_name="s")
    fn = functools.partial(
        pl.kernel,
        mesh=mesh,
        out_type=[
            jax.ShapeDtypeStruct((B, CP), jnp.float32),
            jax.ShapeDtypeStruct((16,), jnp.float32),
        ],
        scratch_types=[
            pltpu.VMEM((8, E), jnp.float32),
            pltpu.VMEM((B, E), jnp.float32),
            pltpu.VMEM((16,), jnp.int32),
            pltpu.VMEM((16, 16), jnp.float32),
            pltpu.VMEM((16, CP), jnp.float32),
            pltpu.VMEM((8, CP), jnp.float32),
            pltpu.VMEM((16,), jnp.float32),
            pltpu.SemaphoreType.DMA,
        ],
    )(_sc_body)
    return fn(logits, oute_flat)


def _conv_bn_relu_pool(h, w, b, gamma, beta, eps=1e-5):
    y = lax.conv_general_dilated(h, w, (1, 1), 'SAME',
                                 dimension_numbers=('NCHW', 'OIHW', 'NCHW'))
    y = y + b[None, :, None, None]
    y = gamma[None, :, None, None] * y / jnp.sqrt(1.0 + eps) + beta[None, :, None, None]
    y = jax.nn.relu(y)
    return lax.reduce_window(y, -jnp.inf, lax.max, (1, 1, 2, 2), (1, 1, 2, 2),
                             'VALID')


def kernel(x, params):
    p = params
    h = x
    for i in range(1, 5):
        h = _conv_bn_relu_pool(h, p['conv%d_w' % i], p['conv%d_b' % i],
                               p['bn%d_g' % i], p['bn%d_b' % i])
    feat = h.reshape(-1, D)

    logits = _gate_logits(feat, p['wg1'], p['bg1'], p['wg2'], p['bg2'])

    w3p = jnp.pad(p['ew3'], ((0, 0), (0, 0), (0, CP - C)))
    b3p = jnp.pad(p['eb3'], ((0, 0), (0, CP - C))).reshape(E, 1, CP)
    oute = _experts(feat,
                    p['ew1'], p['eb1'].reshape(E, 1, H),
                    p['ew2'], p['eb2'].reshape(E, 1, H2),
                    w3p, b3p)

    y_pad, loss_vec = _routing_combine(logits, oute.reshape(E * B, CP))
    return y_pad[:, :C], loss_vec[0].reshape(())


# speedup vs baseline: 2.7189x; 1.0350x over previous
import functools

import jax
import jax.numpy as jnp
from jax import lax
from jax.experimental import pallas as pl
from jax.experimental.pallas import tpu as pltpu
from jax.experimental.pallas import tpu_sc as plsc

E = 16
D = 2048
H = 2048
H2 = 1024
C = 100
CP = 128
B = 256
NK = 4
CK = D // NK
NEG_INF = float('-inf')


def _gate_body(feat_ref, w1_ref, b1_ref, w2_ref, b2_ref, logits_ref, acc_ref):
    i = pl.program_id(0)
    part = jnp.dot(feat_ref[...].astype(jnp.bfloat16),
                   w1_ref[...].astype(jnp.bfloat16),
                   preferred_element_type=jnp.float32)

    @pl.when(i == 0)
    def _():
        acc_ref[...] = part

    @pl.when(i > 0)
    def _():
        acc_ref[...] += part

    @pl.when(i == NK - 1)
    def _():
        g_hid = jnp.maximum(acc_ref[...] + b1_ref[...], 0.0)
        logits_ref[...] = jnp.dot(
            g_hid.astype(jnp.bfloat16), w2_ref[...].astype(jnp.bfloat16),
            preferred_element_type=jnp.float32) + b2_ref[...]


def _gate_logits(feat, wg1, bg1, wg2, bg2):
    return pl.pallas_call(
        _gate_body,
        grid=(NK,),
        in_specs=[
            pl.BlockSpec((B, CK), lambda i: (0, i)),
            pl.BlockSpec((CK, D), lambda i: (i, 0)),
            pl.BlockSpec((1, D), lambda i: (0, 0)),
            pl.BlockSpec((D, E), lambda i: (0, 0)),
            pl.BlockSpec((1, E), lambda i: (0, 0)),
        ],
        out_specs=pl.BlockSpec((B, E), lambda i: (0, 0)),
        out_shape=jax.ShapeDtypeStruct((B, E), jnp.float32),
        scratch_shapes=[pltpu.VMEM((B, D), jnp.float32)],
    )(feat, wg1, bg1.reshape(1, D), wg2, bg2.reshape(1, E))


def _expert_body(feat_ref, gates_ref, w1_ref, b1_ref, w2_ref, b2_ref,
                 w3_ref, b3_ref, y_ref):
    e = pl.program_id(0)
    feat = feat_ref[...].astype(jnp.bfloat16)
    h1 = jnp.dot(feat, w1_ref[0].astype(jnp.bfloat16),
                 preferred_element_type=jnp.float32)
    h1 = jnp.maximum(h1 + b1_ref[0], 0.0).astype(jnp.bfloat16)
    h2 = jnp.dot(h1, w2_ref[0].astype(jnp.bfloat16),
                 preferred_element_type=jnp.float32)
    h2 = jnp.maximum(h2 + b2_ref[0], 0.0).astype(jnp.bfloat16)
    out = jnp.dot(h2, w3_ref[0].astype(jnp.bfloat16),
                  preferred_element_type=jnp.float32)
    out = out + b3_ref[0]
    lane = lax.broadcasted_iota(jnp.int32, (1, E), 1)
    g = jnp.sum(gates_ref[...] * (lane == e).astype(jnp.float32), axis=1,
                keepdims=True)
    contrib = g * out

    @pl.when(e == 0)
    def _init():
        y_ref[...] = contrib

    @pl.when(e > 0)
    def _acc():
        y_ref[...] += contrib


def _experts(feat, gates, w1, b1, w2, b2, w3, b3):
    return pl.pallas_call(
        _expert_body,
        grid=(E,),
        in_specs=[
            pl.BlockSpec((B, D), lambda e: (0, 0)),
            pl.BlockSpec((B, E), lambda e: (0, 0)),
            pl.BlockSpec((1, D, H), lambda e: (e, 0, 0)),
            pl.BlockSpec((1, 1, H), lambda e: (e, 0, 0)),
            pl.BlockSpec((1, H, H2), lambda e: (e, 0, 0)),
            pl.BlockSpec((1, 1, H2), lambda e: (e, 0, 0)),
            pl.BlockSpec((1, H2, C), lambda e: (e, 0, 0)),
            pl.BlockSpec((1, 1, C), lambda e: (e, 0, 0)),
        ],
        out_specs=pl.BlockSpec((B, C), lambda e: (0, 0)),
        out_shape=jax.ShapeDtypeStruct((B, C), jnp.float32),
    )(feat, gates, w1, b1, w2, b2, w3, b3)


def _bfly_max(v, lane):
    for k in (1, 2, 4, 8):
        v = jnp.maximum(v, v.at[lane ^ k].get(mode='promise_in_bounds'))
    return v


def _bfly_min_i32(v, lane):
    for k in (1, 2, 4, 8):
        v = jnp.minimum(v, v.at[lane ^ k].get(mode='promise_in_bounds'))
    return v


def _bfly_sum(v, lane):
    for k in (1, 2, 4, 8):
        v = v + v.at[lane ^ k].get(mode='promise_in_bounds')
    return v


def _lowest_lane(mask, lane):
    return _bfly_min_i32(jnp.where(mask, lane, jnp.int32(64)), lane)


def _top2_row(row, lane):
    m1 = _bfly_max(row, lane)
    i1 = _lowest_lane(row == m1, lane)
    row2 = jnp.where(lane == i1, NEG_INF, row)
    m2 = _bfly_max(row2, lane)
    i2 = _lowest_lane(row2 == m2, lane)
    e2 = jnp.exp(m2 - m1)
    den = 1.0 + e2
    g1 = 1.0 / den
    g2 = e2 / den
    zero = jnp.zeros((16,), jnp.float32)
    gates_row = (jnp.where(lane == i1, g1, zero)
                 + jnp.where(lane == i2, g2, zero))
    return i1, i2, g1, g2, gates_row


def _sc_body(logits_hbm, gates_hbm, loss_hbm,
             lg_v, lg_all, gts_v, loss_v):
    cid = lax.axis_index("c")
    sid = lax.axis_index("s")
    wid = sid * 2 + cid
    base = wid * 8
    lane = lax.iota(jnp.int32, 16)

    pltpu.sync_copy(logits_hbm.at[pl.ds(base, 8)], lg_v)

    for t in range(8):
        row = lg_v[t, pl.ds(0, 16)]
        _, _, _, _, gates_row = _top2_row(row, lane)
        gts_v[t, pl.ds(0, 16)] = gates_row
    pltpu.sync_copy(gts_v, gates_hbm.at[pl.ds(base, 8)])

    @pl.when(jnp.logical_and(cid == 0, sid == 0))
    def _loss():
        pltpu.sync_copy(logits_hbm, lg_all)

        def step(i, carry):
            imp, ld = carry
            row = lg_all[i, pl.ds(0, 16)]
            _, _, _, _, gates_row = _top2_row(row, lane)
            imp = imp + gates_row
            ld = ld + jnp.where(gates_row > 0.0,
                                jnp.full((16,), 1.0, jnp.float32),
                                jnp.zeros((16,), jnp.float32))
            return imp, ld

        zero = jnp.zeros((16,), jnp.float32)
        imp, ld = lax.fori_loop(0, B, step, (zero, zero))

        def cv_sq(v):
            mean = _bfly_sum(v, lane) * (1.0 / 16.0)
            d = v - mean
            var = _bfly_sum(d * d, lane) * (1.0 / 15.0)
            return var / (mean * mean + 1e-10)

        loss = (cv_sq(imp) + cv_sq(ld)) * 1e-2
        loss_v[...] = loss
        pltpu.sync_copy(loss_v, loss_hbm)


def _routing(logits):
    mesh = plsc.VectorSubcoreMesh(core_axis_name="c", subcore_axis_name="s")
    fn = functools.partial(
        pl.kernel,
        mesh=mesh,
        out_type=[
            jax.ShapeDtypeStruct((B, E), jnp.float32),
            jax.ShapeDtypeStruct((16,), jnp.float32),
        ],
        scratch_types=[
            pltpu.VMEM((8, E), jnp.float32),
            pltpu.VMEM((B, E), jnp.float32),
            pltpu.VMEM((8, E), jnp.float32),
            pltpu.VMEM((16,), jnp.float32),
        ],
    )(_sc_body)
    return fn(logits)


def _conv_bn_relu_pool(h, w, b, gamma, beta, eps=1e-5):
    y = lax.conv_general_dilated(h, w, (1, 1), 'SAME',
                                 dimension_numbers=('NCHW', 'OIHW', 'NCHW'))
    y = y + b[None, :, None, None]
    y = gamma[None, :, None, None] * y / jnp.sqrt(1.0 + eps) + beta[None, :, None, None]
    y = jax.nn.relu(y)
    return lax.reduce_window(y, -jnp.inf, lax.max, (1, 1, 2, 2), (1, 1, 2, 2),
                             'VALID')


def kernel(x, params):
    p = params
    h = x
    for i in range(1, 5):
        h = _conv_bn_relu_pool(h, p['conv%d_w' % i], p['conv%d_b' % i],
                               p['bn%d_g' % i], p['bn%d_b' % i])
    feat = h.reshape(-1, D)

    logits = _gate_logits(feat, p['wg1'], p['bg1'], p['wg2'], p['bg2'])

    gates, loss_vec = _routing(logits)
    y = _experts(feat, gates,
                 p['ew1'], p['eb1'].reshape(E, 1, H),
                 p['ew2'], p['eb2'].reshape(E, 1, H2),
                 p['ew3'], p['eb3'].reshape(E, 1, C))
    return y, loss_vec[0].reshape(())


# speedup vs baseline: 2.8772x; 1.0582x over previous
import functools

import jax
import jax.numpy as jnp
from jax import lax
from jax.experimental import pallas as pl
from jax.experimental.pallas import tpu as pltpu
from jax.experimental.pallas import tpu_sc as plsc

E = 16
D = 2048
H = 2048
H2 = 1024
C = 100
B = 256
NK = 4
CK = D // NK
NEG_INF = float('-inf')


def _gate_body(feat_ref, w1_ref, b1_ref, w2_ref, b2_ref, logits_ref, acc_ref):
    i = pl.program_id(0)
    part = jnp.dot(feat_ref[...].astype(jnp.bfloat16),
                   w1_ref[...].astype(jnp.bfloat16),
                   preferred_element_type=jnp.float32)

    @pl.when(i == 0)
    def _():
        acc_ref[...] = part

    @pl.when(i > 0)
    def _():
        acc_ref[...] += part

    @pl.when(i == NK - 1)
    def _():
        g_hid = jnp.maximum(acc_ref[...] + b1_ref[...], 0.0)
        logits_ref[...] = jnp.dot(
            g_hid.astype(jnp.bfloat16), w2_ref[...].astype(jnp.bfloat16),
            preferred_element_type=jnp.float32) + b2_ref[...]


def _gate_logits(feat, wg1, bg1, wg2, bg2):
    return pl.pallas_call(
        _gate_body,
        grid=(NK,),
        in_specs=[
            pl.BlockSpec((B, CK), lambda i: (0, i)),
            pl.BlockSpec((CK, D), lambda i: (i, 0)),
            pl.BlockSpec((1, D), lambda i: (0, 0)),
            pl.BlockSpec((D, E), lambda i: (0, 0)),
            pl.BlockSpec((1, E), lambda i: (0, 0)),
        ],
        out_specs=pl.BlockSpec((B, E), lambda i: (0, 0)),
        out_shape=jax.ShapeDtypeStruct((B, E), jnp.float32),
        scratch_shapes=[pltpu.VMEM((B, D), jnp.float32)],
    )(feat, wg1, bg1.reshape(1, D), wg2, bg2.reshape(1, E))


def _expert_body(feat_ref, gates_ref, w1_ref, b1_ref, w2_ref, b2_ref,
                 w3_ref, b3_ref, y_ref):
    e = pl.program_id(0)
    feat = feat_ref[...].astype(jnp.bfloat16)
    h1 = jnp.dot(feat, w1_ref[0].astype(jnp.bfloat16),
                 preferred_element_type=jnp.float32)
    h1 = jnp.maximum(h1 + b1_ref[0], 0.0).astype(jnp.bfloat16)
    h2 = jnp.dot(h1, w2_ref[0].astype(jnp.bfloat16),
                 preferred_element_type=jnp.float32)
    h2 = jnp.maximum(h2 + b2_ref[0], 0.0).astype(jnp.bfloat16)
    out = jnp.dot(h2, w3_ref[0].astype(jnp.bfloat16),
                  preferred_element_type=jnp.float32)
    out = out + b3_ref[0]
    lane = lax.broadcasted_iota(jnp.int32, (1, E), 1)
    g = jnp.sum(gates_ref[...] * (lane == e).astype(jnp.float32), axis=1,
                keepdims=True)
    contrib = g * out

    @pl.when(e == 0)
    def _init():
        y_ref[...] = contrib

    @pl.when(e > 0)
    def _acc():
        y_ref[...] += contrib


def _experts(feat, gates, w1, b1, w2, b2, w3, b3):
    return pl.pallas_call(
        _expert_body,
        grid=(E,),
        in_specs=[
            pl.BlockSpec((B, D), lambda e: (0, 0)),
            pl.BlockSpec((B, E), lambda e: (0, 0)),
            pl.BlockSpec((1, D, H), lambda e: (e, 0, 0)),
            pl.BlockSpec((1, 1, H), lambda e: (e, 0, 0)),
            pl.BlockSpec((1, H, H2), lambda e: (e, 0, 0)),
            pl.BlockSpec((1, 1, H2), lambda e: (e, 0, 0)),
            pl.BlockSpec((1, H2, C), lambda e: (e, 0, 0)),
            pl.BlockSpec((1, 1, C), lambda e: (e, 0, 0)),
        ],
        out_specs=pl.BlockSpec((B, C), lambda e: (0, 0)),
        out_shape=jax.ShapeDtypeStruct((B, C), jnp.float32),
    )(feat, gates, w1, b1, w2, b2, w3, b3)


def _bfly_max(v, lane):
    for k in (1, 2, 4, 8):
        v = jnp.maximum(v, v.at[lane ^ k].get(mode='promise_in_bounds'))
    return v


def _bfly_min_i32(v, lane):
    for k in (1, 2, 4, 8):
        v = jnp.minimum(v, v.at[lane ^ k].get(mode='promise_in_bounds'))
    return v


def _bfly_sum(v, lane):
    for k in (1, 2, 4, 8):
        v = v + v.at[lane ^ k].get(mode='promise_in_bounds')
    return v


def _lowest_lane(mask, lane):
    return _bfly_min_i32(jnp.where(mask, lane, jnp.int32(64)), lane)


def _top2_row(row, lane):
    m1 = _bfly_max(row, lane)
    i1 = _lowest_lane(row == m1, lane)
    row2 = jnp.where(lane == i1, NEG_INF, row)
    m2 = _bfly_max(row2, lane)
    i2 = _lowest_lane(row2 == m2, lane)
    e2 = jnp.exp(m2 - m1)
    den = 1.0 + e2
    g1 = 1.0 / den
    g2 = e2 / den
    zero = jnp.zeros((16,), jnp.float32)
    gates_row = (jnp.where(lane == i1, g1, zero)
                 + jnp.where(lane == i2, g2, zero))
    return i1, i2, g1, g2, gates_row


def _sc_body(logits_hbm, gates_hbm, loss_hbm,
             lg_v, lg_all, gts_v, loss_v):
    cid = lax.axis_index("c")
    sid = lax.axis_index("s")
    wid = sid * 2 + cid
    base = wid * 8
    lane = lax.iota(jnp.int32, 16)

    pltpu.sync_copy(logits_hbm.at[pl.ds(base, 8)], lg_v)

    for t in range(8):
        row = lg_v[t, pl.ds(0, 16)]
        _, _, _, _, gates_row = _top2_row(row, lane)
        gts_v[t, pl.ds(0, 16)] = gates_row
    pltpu.sync_copy(gts_v, gates_hbm.at[pl.ds(base, 8)])

    @pl.when(jnp.logical_and(cid == 0, sid == 0))
    def _loss():
        pltpu.sync_copy(logits_hbm, lg_all)

        def step(i, carry):
            imp, ld = carry
            row = lg_all[i, pl.ds(0, 16)]
            _, _, _, _, gates_row = _top2_row(row, lane)
            imp = imp + gates_row
            ld = ld + jnp.where(gates_row > 0.0,
                                jnp.full((16,), 1.0, jnp.float32),
                                jnp.zeros((16,), jnp.float32))
            return imp, ld

        zero = jnp.zeros((16,), jnp.float32)
        imp, ld = lax.fori_loop(0, B, step, (zero, zero))

        def cv_sq(v):
            mean = _bfly_sum(v, lane) * (1.0 / 16.0)
            d = v - mean
            var = _bfly_sum(d * d, lane) * (1.0 / 15.0)
            return var / (mean * mean + 1e-10)

        loss = (cv_sq(imp) + cv_sq(ld)) * 1e-2
        loss_v[...] = loss
        pltpu.sync_copy(loss_v, loss_hbm)


def _routing(logits):
    mesh = plsc.VectorSubcoreMesh(core_axis_name="c", subcore_axis_name="s")
    fn = functools.partial(
        pl.kernel,
        mesh=mesh,
        out_type=[
            jax.ShapeDtypeStruct((B, E), jnp.float32),
            jax.ShapeDtypeStruct((16,), jnp.float32),
        ],
        scratch_types=[
            pltpu.VMEM((8, E), jnp.float32),
            pltpu.VMEM((B, E), jnp.float32),
            pltpu.VMEM((8, E), jnp.float32),
            pltpu.VMEM((16,), jnp.float32),
        ],
    )(_sc_body)
    return fn(logits)


def _conv_bn_relu_pool(h, w, b, gamma, beta, eps=1e-5):
    y = lax.conv_general_dilated(h, w, (1, 1), 'SAME',
                                 dimension_numbers=('NCHW', 'OIHW', 'NCHW'))
    y = y + b[None, :, None, None]
    y = gamma[None, :, None, None] * y / jnp.sqrt(1.0 + eps) + beta[None, :, None, None]
    y = jax.nn.relu(y)
    Bn, Co, Hh, Ww = y.shape
    y = y.reshape(Bn, Co, Hh // 2, 2, Ww // 2, 2)
    return y.max(axis=(3, 5))


def kernel(x, params):
    p = params
    h = x
    for i in range(1, 5):
        h = _conv_bn_relu_pool(h, p['conv%d_w' % i], p['conv%d_b' % i],
                               p['bn%d_g' % i], p['bn%d_b' % i])
    feat = h.reshape(-1, D)

    logits = _gate_logits(feat, p['wg1'], p['bg1'], p['wg2'], p['bg2'])

    gates, loss_vec = _routing(logits)
    y = _experts(feat, gates,
                 p['ew1'], p['eb1'].reshape(E, 1, H),
                 p['ew2'], p['eb2'].reshape(E, 1, H2),
                 p['ew3'], p['eb3'].reshape(E, 1, C))
    return y, loss_vec[0].reshape(())
